# Initial kernel scaffold; baseline (speedup 1.0000x reference)
import jax
import jax.numpy as jnp
from jax.experimental import pallas as pl


def kernel(var_node_features, con_node_features, edge_index_var, edge_index_con, edge_features_var, edge_features_con, rhs, asums, rand_var, rand_con, params):
    raise NotImplementedError("write your pallas kernel here")


# speedup vs baseline: 9.9913x; 9.9913x over previous
import functools

import jax
import jax.numpy as jnp
from jax import lax
from jax.experimental import pallas as pl
from jax.experimental.pallas import tpu as pltpu
from jax.experimental.pallas import tpu_sc as plsc

NV = 5000
NC = 5000
E = 160000
D = 128
NROW = 5120
EPAD = 163840
NWORK = 32
NCHUNK = 40
CW = 128
RPT = NROW // 16

_MESH = plsc.VectorSubcoreMesh(core_axis_name="c", subcore_axis_name="s")


def _spmm_body(x_hbm, g_hbm, s_hbm, o_hbm, acc, gi, si, rows, sem):
    cid = lax.axis_index("c")
    sid = lax.axis_index("s")
    w = cid * 16 + sid
    base = sid * RPT

    @pl.loop(0, CW)
    def _zr(r):
        @pl.loop(0, 128, step=16)
        def _zc(cc):
            rows[r, pl.ds(cc, 16)] = jnp.zeros((16,), jnp.float32)

    pltpu.sync_copy(rows, acc.at[pl.ds(base, 128)])
    pltpu.sync_copy(rows, acc.at[pl.ds(base + 128, 128)])
    pltpu.sync_copy(rows.at[pl.ds(0, 64)], acc.at[pl.ds(base + 256, 64)])
    plsc.subcore_barrier()

    pltpu.sync_copy(g_hbm.at[w], gi)
    pltpu.sync_copy(s_hbm.at[w], si)

    @pl.loop(0, NCHUNK)
    def _ch(j):
        pltpu.async_copy(x_hbm.at[gi.at[j]], rows, sem).wait()
        pltpu.sync_copy(rows, acc.at[si.at[j]], add=True)

    plsc.subcore_barrier()

    for off, nn in ((0, 128), (128, 128), (256, 64)):
        pltpu.sync_copy(acc.at[pl.ds(base + off, nn)], rows.at[pl.ds(0, nn)])
        pltpu.sync_copy(rows.at[pl.ds(0, nn)], o_hbm.at[cid, pl.ds(base + off, nn)])


def _sc_spmm(x, gidx, sidx):
    k = pl.kernel(
        _spmm_body,
        out_type=jax.ShapeDtypeStruct((2, NROW, 128), jnp.float32),
        mesh=_MESH,
        scratch_types=[
            pltpu.VMEM_SHARED((NROW, 128), jnp.float32),
            pltpu.VMEM((NCHUNK, CW), jnp.int32),
            pltpu.VMEM((NCHUNK, CW), jnp.int32),
            pltpu.VMEM((CW, 128), jnp.float32),
            pltpu.SemaphoreType.DMA,
        ],
    )
    return k(x, gidx, sidx)


def _deg_body(g_hbm, s_hbm, ov_hbm, oc_hbm, accv, accc, gi, si, ones, buf, sem):
    cid = lax.axis_index("c")
    sid = lax.axis_index("s")
    base = sid * RPT

    @pl.loop(0, CW, step=16)
    def _z(cc):
        ones[pl.ds(cc, 16)] = jnp.ones((16,), jnp.float32)
        buf[pl.ds(cc, 16)] = jnp.zeros((16,), jnp.float32)

    for accr in (accv, accc):
        pltpu.sync_copy(buf, accr.at[pl.ds(base, 128)])
        pltpu.sync_copy(buf, accr.at[pl.ds(base + 128, 128)])
        pltpu.sync_copy(buf.at[pl.ds(0, 64)], accr.at[pl.ds(base + 256, 64)])
    plsc.subcore_barrier()

    for blk in range(2):
        w = blk * 16 + sid
        pltpu.sync_copy(g_hbm.at[w], gi)
        pltpu.sync_copy(s_hbm.at[w], si)

        @pl.loop(0, NCHUNK)
        def _ch(j):
            pltpu.sync_copy(ones, accv.at[gi.at[j]], add=True)
            pltpu.sync_copy(ones, accc.at[si.at[j]], add=True)

    plsc.subcore_barrier()

    for accr, out in ((accv, ov_hbm), (accc, oc_hbm)):
        for off, nn in ((0, 128), (128, 128), (256, 64)):
            pltpu.sync_copy(accr.at[pl.ds(base + off, nn)], buf.at[pl.ds(0, nn)])

            @pl.loop(0, nn, step=16)
            def _r(cc):
                dv = buf[pl.ds(cc, 16)]
                buf[pl.ds(cc, 16)] = jnp.where(
                    dv > 0.0, 1.0 / jnp.where(dv > 0.0, dv, 1.0), 0.0)

            @pl.when(cid == 0)
            def _w():
                pltpu.sync_copy(buf.at[pl.ds(0, nn)], out.at[pl.ds(base + off, nn)])


def _sc_degrees(gidx, sidx):
    k = pl.kernel(
        _deg_body,
        out_type=(jax.ShapeDtypeStruct((NROW,), jnp.float32),
                  jax.ShapeDtypeStruct((NROW,), jnp.float32)),
        mesh=_MESH,
        scratch_types=[
            pltpu.VMEM_SHARED((NROW,), jnp.float32),
            pltpu.VMEM_SHARED((NROW,), jnp.float32),
            pltpu.VMEM((NCHUNK, CW), jnp.int32),
            pltpu.VMEM((NCHUNK, CW), jnp.int32),
            pltpu.VMEM((CW,), jnp.float32),
            pltpu.VMEM((CW,), jnp.float32),
            pltpu.SemaphoreType.DMA,
        ],
    )
    return k(gidx, sidx)


def _lane_mask():
    return lax.broadcasted_iota(jnp.int32, (NROW, 128), 1) == 127


def _init_body(vfp_ref, w1_ref, b1_ref, w2_ref, b2_ref, m_ref):
    h = jnp.maximum(vfp_ref[...] @ w1_ref[...] + b1_ref[...][None, :], 0.0)
    m_ref[...] = h @ w2_ref[...] + b2_ref[...][None, :]


def _tc_init(vfp, w1, b1, w2, b2):
    return pl.pallas_call(
        _init_body,
        out_shape=jax.ShapeDtypeStruct((NROW, 128), jnp.float32),
    )(vfp, w1, b1, w2, b2)


def _prep_body(vh_ref, cons_ref, dinv_ref, efv_ref, rhs_ref,
               w1h_ref, b1h_ref, w2h_ref, b2h_ref,
               w1m_ref, b1m_ref, w2m_ref, b2m_ref,
               rootv_ref, biasv_ref,
               xv_ref, h_ref, r1_ref):
    mask = _lane_mask()
    vh = vh_ref[...]
    h1 = jnp.maximum(vh @ w1h_ref[...] + b1h_ref[...][None, :], 0.0)
    H = jnp.sum(h1 * w2h_ref[...][None, :], axis=1, keepdims=True) + b2h_ref[0, 0]
    m1 = jnp.maximum(vh @ w1m_ref[...] + b1m_ref[...][None, :], 0.0)
    M = m1 @ w2m_ref[...] + b2m_ref[...][None, :]
    xv_ref[...] = jnp.where(mask, H * efv_ref[...], dinv_ref[...] * M)
    h_ref[...] = H
    r1_ref[...] = (cons_ref[...] @ rootv_ref[...] + biasv_ref[...][None, :]
                   - jnp.where(mask, rhs_ref[...], 0.0))


def _tc_prep(vh, cons, dinv_v, efv, rhs, wts):
    return pl.pallas_call(
        _prep_body,
        out_shape=(jax.ShapeDtypeStruct((NROW, 128), jnp.float32),
                   jax.ShapeDtypeStruct((NROW, 1), jnp.float32),
                   jax.ShapeDtypeStruct((NROW, 128), jnp.float32)),
    )(vh, cons, dinv_v, efv, rhs, *wts)


def _updc_body(p0_ref, p1_ref, r1_ref, vh_ref, efc_ref, asums_ref, dinv_ref,
               w1c_ref, b1c_ref, w2c_ref, b2c_ref, rootc_ref, biasc_ref,
               cons_ref, xc_ref, r2_ref):
    mask = _lane_mask()
    cons = jnp.maximum(p0_ref[...] + p1_ref[...] + r1_ref[...], 0.0)
    last = jnp.sum(jnp.where(mask, cons, 0.0), axis=1, keepdims=True)
    m1 = jnp.maximum(cons @ w1c_ref[...] + b1c_ref[...][None, :], 0.0)
    Mc = m1 @ w2c_ref[...] + b2c_ref[...][None, :]
    a = efc_ref[...] / asums_ref[...] * last
    xc_ref[...] = dinv_ref[...] * jnp.where(mask, a, Mc)
    cons_ref[...] = cons
    r2_ref[...] = vh_ref[...] @ rootc_ref[...] + biasc_ref[...][None, :]


def _tc_updc(p0, p1, r1, vh, efc, asums, dinv_c, wts):
    return pl.pallas_call(
        _updc_body,
        out_shape=(jax.ShapeDtypeStruct((NROW, 128), jnp.float32),
                   jax.ShapeDtypeStruct((NROW, 128), jnp.float32),
                   jax.ShapeDtypeStruct((NROW, 128), jnp.float32)),
    )(p0, p1, r1, vh, efc, asums, dinv_c, *wts)


def _updv_body(q0_ref, q1_ref, r2_ref, h_ref, vnew_ref):
    mask = _lane_mask()
    S = q0_ref[...] + q1_ref[...]
    last = jnp.sum(jnp.where(mask, S, 0.0), axis=1, keepdims=True) * h_ref[...]
    vnew_ref[...] = jnp.maximum(jnp.where(mask, last, S + r2_ref[...]), 0.0)


def _tc_updv(q0, q1, r2, h):
    return pl.pallas_call(
        _updv_body,
        out_shape=jax.ShapeDtypeStruct((NROW, 128), jnp.float32),
    )(q0, q1, r2, h)


def _final_body(q0_ref, q1_ref, r2_ref, h_ref,
                w1_ref, b1_ref, w2_ref, b2_ref, w3_ref, b3_ref,
                w4_ref, b4_ref, w5_ref, b5_ref, w6_ref, b6_ref, out_ref):
    mask = _lane_mask()
    S = q0_ref[...] + q1_ref[...]
    last = jnp.sum(jnp.where(mask, S, 0.0), axis=1, keepdims=True) * h_ref[...]
    x = jnp.maximum(jnp.where(mask, last, S + r2_ref[...]), 0.0)
    for w_ref, b_ref in ((w1_ref, b1_ref), (w2_ref, b2_ref), (w3_ref, b3_ref),
                         (w4_ref, b4_ref), (w5_ref, b5_ref)):
        x = jnp.maximum(x @ w_ref[...] + b_ref[...][None, :], 0.0)
    logit = jnp.sum(x * w6_ref[...][None, :], axis=1, keepdims=True) + b6_ref[0, 0]
    out_ref[...] = jax.nn.sigmoid(logit)


def _tc_final(q0, q1, r2, h, fcw):
    return pl.pallas_call(
        _final_body,
        out_shape=jax.ShapeDtypeStruct((NROW, 1), jnp.float32),
    )(q0, q1, r2, h, *fcw)


def _padw(w, rows=128, cols=128):
    r, c = w.shape
    return jnp.pad(w, ((0, rows - r), (0, cols - c)))


def _padb(b, n=128):
    return jnp.pad(b, (0, n - b.shape[0]))


def _padcol(x, n=NROW):
    return jnp.pad(x, ((0, n - x.shape[0]), (0, 0)))


def _layer_wts_v2c(lp):
    h2v, mm = lp["h2v"], lp["v2c_mlp"]
    return (_padw(h2v["l1"]["W"]), _padb(h2v["l1"]["b"]),
            _padb(h2v["l2"]["W"][:, 0]), h2v["l2"]["b"].reshape(1, 1),
            _padw(mm["l1"]["W"]), _padb(mm["l1"]["b"]),
            _padw(mm["l2"]["W"]), _padb(mm["l2"]["b"]),
            _padw(lp["v2c_root"]), _padb(lp["v2c_bias"]))


def _layer_wts_c2v(lp):
    mm = lp["c2v_mlp"]
    return (_padw(mm["l1"]["W"]), _padb(mm["l1"]["b"]),
            _padw(mm["l2"]["W"]), _padb(mm["l2"]["b"]),
            _padw(lp["c2v_root"]), _padb(lp["c2v_bias"]))


def kernel(var_node_features, con_node_features, edge_index_var, edge_index_con,
           edge_features_var, edge_features_con, rhs, asums, rand_var, rand_con,
           params):
    f32 = jnp.float32
    src = edge_index_var[0]
    dst = edge_index_var[1]

    npad = EPAD - E
    spread = jnp.arange(npad, dtype=jnp.int32) % (NROW - NV)
    srcp = jnp.concatenate([src, NV + spread]).reshape(NWORK, NCHUNK, CW)
    dstp = jnp.concatenate([dst, NC + spread]).reshape(NWORK, NCHUNK, CW)

    dinv_v1, dinv_c1 = _sc_degrees(srcp, dstp)
    dinv_v = dinv_v1.reshape(NROW, 1)
    dinv_c = dinv_c1.reshape(NROW, 1)

    vfp = jnp.pad(var_node_features, ((0, NROW - NV), (0, 126)))
    cfp = jnp.pad(con_node_features, ((0, NROW - NC), (0, 126)))
    vm, cm = params["var_mlp"], params["con_mlp"]
    Mv = _tc_init(vfp, _padw(vm["l1"]["W"]), _padb(vm["l1"]["b"]),
                  _padw(vm["l2"]["W"]), _padb(vm["l2"]["b"]))
    Mc0 = _tc_init(cfp, _padw(cm["l1"]["W"]), _padb(cm["l1"]["b"]),
                   _padw(cm["l2"]["W"]), _padb(cm["l2"]["b"]))

    zero1 = jnp.zeros((NROW, 1), f32)
    var_h = jnp.concatenate(
        [_padcol(rand_var), Mv[:, :61], vfp[:, :2], zero1], axis=1)
    cons = jnp.concatenate(
        [_padcol(rand_con), Mc0[:, :61], cfp[:, :2], zero1], axis=1)

    efv = _padcol(edge_features_var[:NV])
    efc = _padcol(edge_features_con[:NC])
    rhsp = jnp.pad(rhs, (0, NROW - NC)).reshape(NROW, 1)
    asumsp = jnp.concatenate([asums, jnp.ones((NROW - NC,), f32)]).reshape(NROW, 1)

    fc = (_padw(params["fc1"]["W"]), _padb(params["fc1"]["b"]),
          _padw(params["fc2"]["W"]), _padb(params["fc2"]["b"]),
          _padw(params["fc3"]["W"]), _padb(params["fc3"]["b"]),
          _padw(params["fc4"]["W"]), _padb(params["fc4"]["b"]),
          _padw(params["fc5"]["W"]), _padb(params["fc5"]["b"]),
          _padb(params["fc6"]["W"][:, 0]), params["fc6"]["b"].reshape(1, 1))

    out = None
    for i in range(1, 7):
        lp = params["layer%d" % i]
        xv, H, r1 = _tc_prep(var_h, cons, dinv_v, efv, rhsp, _layer_wts_v2c(lp))
        p = _sc_spmm(xv, srcp, dstp)
        cons, xc, r2 = _tc_updc(p[0], p[1], r1, var_h, efc, asumsp, dinv_c,
                                _layer_wts_c2v(lp))
        q = _sc_spmm(xc, dstp, srcp)
        if i < 6:
            var_h = _tc_updv(q[0], q[1], r2, H)
        else:
            out = _tc_final(q[0], q[1], r2, H, fc)

    return out[:NV]


# speedup vs baseline: 14.3228x; 1.4335x over previous
import functools

import jax
import jax.numpy as jnp
from jax import lax
from jax.experimental import pallas as pl
from jax.experimental.pallas import tpu as pltpu
from jax.experimental.pallas import tpu_sc as plsc

NV = 5000
NC = 5000
E = 160000
D = 128
NROW = 5120
EPAD = 163840
NWORK = 32
NCHUNK = 40
CW = 128
RPT = NROW // 16

_MESH = plsc.VectorSubcoreMesh(core_axis_name="c", subcore_axis_name="s")


def _spmm_body(x_hbm, g_hbm, s_hbm, o_hbm, acc, gi, si, rows0, rows1, sem0, sem1):
    cid = lax.axis_index("c")
    sid = lax.axis_index("s")
    w = cid * 16 + sid
    base = sid * RPT

    @pl.loop(0, CW)
    def _zr(r):
        @pl.loop(0, 128, step=16)
        def _zc(cc):
            rows0[r, pl.ds(cc, 16)] = jnp.zeros((16,), jnp.float32)

    pltpu.sync_copy(rows0, acc.at[pl.ds(base, 128)])
    pltpu.sync_copy(rows0, acc.at[pl.ds(base + 128, 128)])
    pltpu.sync_copy(rows0.at[pl.ds(0, 64)], acc.at[pl.ds(base + 256, 64)])
    plsc.subcore_barrier()

    pltpu.sync_copy(g_hbm.at[w], gi)
    pltpu.sync_copy(s_hbm.at[w], si)

    pltpu.async_copy(x_hbm.at[gi.at[0]], rows0, sem0)

    @pl.loop(0, NCHUNK, step=2)
    def _ch(j):
        pltpu.async_copy(x_hbm.at[gi.at[j + 1]], rows1, sem1)
        pltpu.make_async_copy(x_hbm.at[gi.at[j]], rows0, sem0).wait()
        pltpu.sync_copy(rows0, acc.at[si.at[j]], add=True)

        @pl.when(j + 2 < NCHUNK)
        def _pf():
            pltpu.async_copy(x_hbm.at[gi.at[j + 2]], rows0, sem0)

        pltpu.make_async_copy(x_hbm.at[gi.at[j + 1]], rows1, sem1).wait()
        pltpu.sync_copy(rows1, acc.at[si.at[j + 1]], add=True)

    plsc.subcore_barrier()

    for off, nn in ((0, 128), (128, 128), (256, 64)):
        pltpu.sync_copy(acc.at[pl.ds(base + off, nn)], rows0.at[pl.ds(0, nn)])
        pltpu.sync_copy(rows0.at[pl.ds(0, nn)], o_hbm.at[cid, pl.ds(base + off, nn)])


def _sc_spmm(x, gidx, sidx):
    k = pl.kernel(
        _spmm_body,
        out_type=jax.ShapeDtypeStruct((2, NROW, 128), jnp.float32),
        mesh=_MESH,
        scratch_types=[
            pltpu.VMEM_SHARED((NROW, 128), jnp.float32),
            pltpu.VMEM((NCHUNK, CW), jnp.int32),
            pltpu.VMEM((NCHUNK, CW), jnp.int32),
            pltpu.VMEM((CW, 128), jnp.float32),
            pltpu.VMEM((CW, 128), jnp.float32),
            pltpu.SemaphoreType.DMA,
            pltpu.SemaphoreType.DMA,
        ],
    )
    return k(x, gidx, sidx)


def _deg_body(g_hbm, s_hbm, ov_hbm, oc_hbm, accv, accc, gi, si, ones, buf, sem):
    cid = lax.axis_index("c")
    sid = lax.axis_index("s")
    base = sid * RPT

    @pl.loop(0, CW, step=16)
    def _z(cc):
        ones[pl.ds(cc, 16)] = jnp.ones((16,), jnp.float32)
        buf[pl.ds(cc, 16)] = jnp.zeros((16,), jnp.float32)

    for accr in (accv, accc):
        pltpu.sync_copy(buf, accr.at[pl.ds(base, 128)])
        pltpu.sync_copy(buf, accr.at[pl.ds(base + 128, 128)])
        pltpu.sync_copy(buf.at[pl.ds(0, 64)], accr.at[pl.ds(base + 256, 64)])
    plsc.subcore_barrier()

    for blk in range(2):
        w = blk * 16 + sid
        pltpu.sync_copy(g_hbm.at[w], gi)
        pltpu.sync_copy(s_hbm.at[w], si)

        @pl.loop(0, NCHUNK)
        def _ch(j):
            pltpu.sync_copy(ones, accv.at[gi.at[j]], add=True)
            pltpu.sync_copy(ones, accc.at[si.at[j]], add=True)

    plsc.subcore_barrier()

    for accr, out in ((accv, ov_hbm), (accc, oc_hbm)):
        for off, nn in ((0, 128), (128, 128), (256, 64)):
            pltpu.sync_copy(accr.at[pl.ds(base + off, nn)], buf.at[pl.ds(0, nn)])

            @pl.loop(0, nn, step=16)
            def _r(cc):
                dv = buf[pl.ds(cc, 16)]
                buf[pl.ds(cc, 16)] = jnp.where(
                    dv > 0.0, 1.0 / jnp.where(dv > 0.0, dv, 1.0), 0.0)

            @pl.when(cid == 0)
            def _w():
                pltpu.sync_copy(buf.at[pl.ds(0, nn)], out.at[pl.ds(base + off, nn)])


def _sc_degrees(gidx, sidx):
    k = pl.kernel(
        _deg_body,
        out_type=(jax.ShapeDtypeStruct((NROW,), jnp.float32),
                  jax.ShapeDtypeStruct((NROW,), jnp.float32)),
        mesh=_MESH,
        scratch_types=[
            pltpu.VMEM_SHARED((NROW,), jnp.float32),
            pltpu.VMEM_SHARED((NROW,), jnp.float32),
            pltpu.VMEM((NCHUNK, CW), jnp.int32),
            pltpu.VMEM((NCHUNK, CW), jnp.int32),
            pltpu.VMEM((CW,), jnp.float32),
            pltpu.VMEM((CW,), jnp.float32),
            pltpu.SemaphoreType.DMA,
        ],
    )
    return k(gidx, sidx)


def _lane_mask():
    return lax.broadcasted_iota(jnp.int32, (NROW, 128), 1) == 127


def _init_body(vfp_ref, w1_ref, b1_ref, w2_ref, b2_ref, m_ref):
    h = jnp.maximum(vfp_ref[...] @ w1_ref[...] + b1_ref[...][None, :], 0.0)
    m_ref[...] = h @ w2_ref[...] + b2_ref[...][None, :]


def _tc_init(vfp, w1, b1, w2, b2):
    return pl.pallas_call(
        _init_body,
        out_shape=jax.ShapeDtypeStruct((NROW, 128), jnp.float32),
    )(vfp, w1, b1, w2, b2)


def _prep_body(vh_ref, cons_ref, dinv_ref, efv_ref, rhs_ref,
               w1h_ref, b1h_ref, w2h_ref, b2h_ref,
               w1m_ref, b1m_ref, w2m_ref, b2m_ref,
               rootv_ref, biasv_ref,
               xv_ref, h_ref, r1_ref):
    mask = _lane_mask()
    vh = vh_ref[...]
    h1 = jnp.maximum(vh @ w1h_ref[...] + b1h_ref[...][None, :], 0.0)
    H = jnp.sum(h1 * w2h_ref[...][None, :], axis=1, keepdims=True) + b2h_ref[0, 0]
    m1 = jnp.maximum(vh @ w1m_ref[...] + b1m_ref[...][None, :], 0.0)
    M = m1 @ w2m_ref[...] + b2m_ref[...][None, :]
    xv_ref[...] = jnp.where(mask, H * efv_ref[...], dinv_ref[...] * M)
    h_ref[...] = H
    r1_ref[...] = (cons_ref[...] @ rootv_ref[...] + biasv_ref[...][None, :]
                   - jnp.where(mask, rhs_ref[...], 0.0))


def _tc_prep(vh, cons, dinv_v, efv, rhs, wts):
    return pl.pallas_call(
        _prep_body,
        out_shape=(jax.ShapeDtypeStruct((NROW, 128), jnp.float32),
                   jax.ShapeDtypeStruct((NROW, 1), jnp.float32),
                   jax.ShapeDtypeStruct((NROW, 128), jnp.float32)),
    )(vh, cons, dinv_v, efv, rhs, *wts)


def _updc_body(p0_ref, p1_ref, r1_ref, vh_ref, efc_ref, asums_ref, dinv_ref,
               w1c_ref, b1c_ref, w2c_ref, b2c_ref, rootc_ref, biasc_ref,
               cons_ref, xc_ref, r2_ref):
    mask = _lane_mask()
    cons = jnp.maximum(p0_ref[...] + p1_ref[...] + r1_ref[...], 0.0)
    last = jnp.sum(jnp.where(mask, cons, 0.0), axis=1, keepdims=True)
    m1 = jnp.maximum(cons @ w1c_ref[...] + b1c_ref[...][None, :], 0.0)
    Mc = m1 @ w2c_ref[...] + b2c_ref[...][None, :]
    a = efc_ref[...] / asums_ref[...] * last
    xc_ref[...] = dinv_ref[...] * jnp.where(mask, a, Mc)
    cons_ref[...] = cons
    r2_ref[...] = vh_ref[...] @ rootc_ref[...] + biasc_ref[...][None, :]


def _tc_updc(p0, p1, r1, vh, efc, asums, dinv_c, wts):
    return pl.pallas_call(
        _updc_body,
        out_shape=(jax.ShapeDtypeStruct((NROW, 128), jnp.float32),
                   jax.ShapeDtypeStruct((NROW, 128), jnp.float32),
                   jax.ShapeDtypeStruct((NROW, 128), jnp.float32)),
    )(p0, p1, r1, vh, efc, asums, dinv_c, *wts)


def _prepf_body(q0_ref, q1_ref, r2_ref, hp_ref, cons_ref, dinv_ref, efv_ref,
                rhs_ref,
                w1h_ref, b1h_ref, w2h_ref, b2h_ref,
                w1m_ref, b1m_ref, w2m_ref, b2m_ref,
                rootv_ref, biasv_ref,
                xv_ref, h_ref, r1_ref, vh_ref):
    mask = _lane_mask()
    S = q0_ref[...] + q1_ref[...]
    lastp = jnp.sum(jnp.where(mask, S, 0.0), axis=1, keepdims=True) * hp_ref[...]
    vh = jnp.maximum(jnp.where(mask, lastp, S + r2_ref[...]), 0.0)
    h1 = jnp.maximum(vh @ w1h_ref[...] + b1h_ref[...][None, :], 0.0)
    H = jnp.sum(h1 * w2h_ref[...][None, :], axis=1, keepdims=True) + b2h_ref[0, 0]
    m1 = jnp.maximum(vh @ w1m_ref[...] + b1m_ref[...][None, :], 0.0)
    M = m1 @ w2m_ref[...] + b2m_ref[...][None, :]
    xv_ref[...] = jnp.where(mask, H * efv_ref[...], dinv_ref[...] * M)
    h_ref[...] = H
    r1_ref[...] = (cons_ref[...] @ rootv_ref[...] + biasv_ref[...][None, :]
                   - jnp.where(mask, rhs_ref[...], 0.0))
    vh_ref[...] = vh


def _tc_prepf(q0, q1, r2, hp, cons, dinv_v, efv, rhs, wts):
    return pl.pallas_call(
        _prepf_body,
        out_shape=(jax.ShapeDtypeStruct((NROW, 128), jnp.float32),
                   jax.ShapeDtypeStruct((NROW, 1), jnp.float32),
                   jax.ShapeDtypeStruct((NROW, 128), jnp.float32),
                   jax.ShapeDtypeStruct((NROW, 128), jnp.float32)),
    )(q0, q1, r2, hp, cons, dinv_v, efv, rhs, *wts)


def _final_body(q0_ref, q1_ref, r2_ref, h_ref,
                w1_ref, b1_ref, w2_ref, b2_ref, w3_ref, b3_ref,
                w4_ref, b4_ref, w5_ref, b5_ref, w6_ref, b6_ref, out_ref):
    mask = _lane_mask()
    S = q0_ref[...] + q1_ref[...]
    last = jnp.sum(jnp.where(mask, S, 0.0), axis=1, keepdims=True) * h_ref[...]
    x = jnp.maximum(jnp.where(mask, last, S + r2_ref[...]), 0.0)
    for w_ref, b_ref in ((w1_ref, b1_ref), (w2_ref, b2_ref), (w3_ref, b3_ref),
                         (w4_ref, b4_ref), (w5_ref, b5_ref)):
        x = jnp.maximum(x @ w_ref[...] + b_ref[...][None, :], 0.0)
    logit = jnp.sum(x * w6_ref[...][None, :], axis=1, keepdims=True) + b6_ref[0, 0]
    out_ref[...] = jax.nn.sigmoid(logit)


def _tc_final(q0, q1, r2, h, fcw):
    return pl.pallas_call(
        _final_body,
        out_shape=jax.ShapeDtypeStruct((NROW, 1), jnp.float32),
    )(q0, q1, r2, h, *fcw)


def _padw(w, rows=128, cols=128):
    r, c = w.shape
    return jnp.pad(w, ((0, rows - r), (0, cols - c)))


def _padb(b, n=128):
    return jnp.pad(b, (0, n - b.shape[0]))


def _padcol(x, n=NROW):
    return jnp.pad(x, ((0, n - x.shape[0]), (0, 0)))


def _layer_wts_v2c(lp):
    h2v, mm = lp["h2v"], lp["v2c_mlp"]
    return (_padw(h2v["l1"]["W"]), _padb(h2v["l1"]["b"]),
            _padb(h2v["l2"]["W"][:, 0]), h2v["l2"]["b"].reshape(1, 1),
            _padw(mm["l1"]["W"]), _padb(mm["l1"]["b"]),
            _padw(mm["l2"]["W"]), _padb(mm["l2"]["b"]),
            _padw(lp["v2c_root"]), _padb(lp["v2c_bias"]))


def _layer_wts_c2v(lp):
    mm = lp["c2v_mlp"]
    return (_padw(mm["l1"]["W"]), _padb(mm["l1"]["b"]),
            _padw(mm["l2"]["W"]), _padb(mm["l2"]["b"]),
            _padw(lp["c2v_root"]), _padb(lp["c2v_bias"]))


def kernel(var_node_features, con_node_features, edge_index_var, edge_index_con,
           edge_features_var, edge_features_con, rhs, asums, rand_var, rand_con,
           params):
    f32 = jnp.float32
    src = edge_index_var[0]
    dst = edge_index_var[1]

    npad = EPAD - E
    spread = jnp.arange(npad, dtype=jnp.int32) % (NROW - NV)
    srcp = jnp.concatenate([src, NV + spread]).reshape(NWORK, NCHUNK, CW)
    dstp = jnp.concatenate([dst, NC + spread]).reshape(NWORK, NCHUNK, CW)

    dinv_v1, dinv_c1 = _sc_degrees(srcp, dstp)
    dinv_v = dinv_v1.reshape(NROW, 1)
    dinv_c = dinv_c1.reshape(NROW, 1)

    vfp = jnp.pad(var_node_features, ((0, NROW - NV), (0, 126)))
    cfp = jnp.pad(con_node_features, ((0, NROW - NC), (0, 126)))
    vm, cm = params["var_mlp"], params["con_mlp"]
    Mv = _tc_init(vfp, _padw(vm["l1"]["W"]), _padb(vm["l1"]["b"]),
                  _padw(vm["l2"]["W"]), _padb(vm["l2"]["b"]))
    Mc0 = _tc_init(cfp, _padw(cm["l1"]["W"]), _padb(cm["l1"]["b"]),
                   _padw(cm["l2"]["W"]), _padb(cm["l2"]["b"]))

    zero1 = jnp.zeros((NROW, 1), f32)
    var_h = jnp.concatenate(
        [_padcol(rand_var), Mv[:, :61], vfp[:, :2], zero1], axis=1)
    cons = jnp.concatenate(
        [_padcol(rand_con), Mc0[:, :61], cfp[:, :2], zero1], axis=1)

    efv = _padcol(edge_features_var[:NV])
    efc = _padcol(edge_features_con[:NC])
    rhsp = jnp.pad(rhs, (0, NROW - NC)).reshape(NROW, 1)
    asumsp = jnp.concatenate([asums, jnp.ones((NROW - NC,), f32)]).reshape(NROW, 1)

    fc = (_padw(params["fc1"]["W"]), _padb(params["fc1"]["b"]),
          _padw(params["fc2"]["W"]), _padb(params["fc2"]["b"]),
          _padw(params["fc3"]["W"]), _padb(params["fc3"]["b"]),
          _padw(params["fc4"]["W"]), _padb(params["fc4"]["b"]),
          _padw(params["fc5"]["W"]), _padb(params["fc5"]["b"]),
          _padb(params["fc6"]["W"][:, 0]), params["fc6"]["b"].reshape(1, 1))

    out = None
    q = r2 = H = None
    for i in range(1, 7):
        lp = params["layer%d" % i]
        if i == 1:
            xv, H, r1 = _tc_prep(var_h, cons, dinv_v, efv, rhsp,
                                 _layer_wts_v2c(lp))
        else:
            xv, H, r1, var_h = _tc_prepf(q[0], q[1], r2, H, cons, dinv_v, efv,
                                         rhsp, _layer_wts_v2c(lp))
        p = _sc_spmm(xv, srcp, dstp)
        cons, xc, r2 = _tc_updc(p[0], p[1], r1, var_h, efc, asumsp, dinv_c,
                                _layer_wts_c2v(lp))
        q = _sc_spmm(xc, dstp, srcp)
        if i == 6:
            out = _tc_final(q[0], q[1], r2, H, fc)

    return out[:NV]
